# Initial kernel scaffold; baseline (speedup 1.0000x reference)
import jax
import jax.numpy as jnp
from jax.experimental import pallas as pl


def kernel(x, edge_index, edge_attr, W_start, b_start, att_l, att_r, W_end, b_end):
    raise NotImplementedError("write your pallas kernel here")


# speedup vs baseline: 14.3441x; 14.3441x over previous
import functools

import jax
import jax.numpy as jnp
from jax import lax
from jax.experimental import pallas as pl
from jax.experimental.pallas import tpu as pltpu
from jax.experimental.pallas import tpu_sc as plsc

N = 10000
F = 128
FH = F // 2
NCLASS = 16
E = 320000
EPS = 0.1

NC = 2
NS = 16
EPT = E // NS
CH = 80
NCHUNK = EPT // CH
GPC = CH // 16
NRB = N // CH
RBPT = -(-NRB // NS)

ROWBLK = 1000


def _split(h):
    return jnp.stack([h[:, :FH], h[:, FH:]], axis=0)


def _alar(h, att2):
    return jnp.dot(h, att2)


def _tc_start_body(x_ref, wt_ref, b_ref, att2_ref, hs_ref, alar_ref):
    h = jnp.dot(x_ref[...], wt_ref[...], preferred_element_type=jnp.float32)
    h = jnp.maximum(h + b_ref[...], 0.0)
    hs_ref[...] = _split(h)
    alar_ref[...] = _alar(h, att2_ref[...])


def _tc_mid_body(aggs_ref, x0s_ref, att2_ref, hs_ref, alar_ref):
    hs = aggs_ref[...] + EPS * x0s_ref[...]
    hs_ref[...] = hs
    h = jnp.concatenate([hs[0], hs[1]], axis=1)
    alar_ref[...] = _alar(h, att2_ref[...])


def _tc_end_body(aggs_ref, x0s_ref, wt_ref, b_ref, out_ref):
    hs = aggs_ref[...] + EPS * x0s_ref[...]
    h = jnp.concatenate([hs[0], hs[1]], axis=1)
    o = jnp.dot(h, wt_ref[...], preferred_element_type=jnp.float32)
    out_ref[...] = o + b_ref[...]


_HS_SPEC = pl.BlockSpec((NC, ROWBLK, FH), lambda i: (0, i, 0))
_HS_SHAPE = jax.ShapeDtypeStruct((NC, N, FH), jnp.float32)
_ALAR_SPEC = pl.BlockSpec((ROWBLK, 2), lambda i: (i, 0))
_ALAR_SHAPE = jax.ShapeDtypeStruct((N, 2), jnp.float32)
_VEC_SPEC = pl.BlockSpec((1, F), lambda i: (0, 0))
_ATT2_SPEC = pl.BlockSpec((F, 2), lambda i: (0, 0))


def _tc_start(x, wt, b, att2):
    return pl.pallas_call(
        _tc_start_body,
        grid=(N // ROWBLK,),
        in_specs=[
            pl.BlockSpec((ROWBLK, F), lambda i: (i, 0)),
            pl.BlockSpec((F, F), lambda i: (0, 0)),
            _VEC_SPEC, _ATT2_SPEC,
        ],
        out_specs=[_HS_SPEC, _ALAR_SPEC],
        out_shape=[_HS_SHAPE, _ALAR_SHAPE],
    )(x, wt, b, att2)


def _tc_mid(aggs, x0s, att2):
    return pl.pallas_call(
        _tc_mid_body,
        grid=(N // ROWBLK,),
        in_specs=[_HS_SPEC, _HS_SPEC, _ATT2_SPEC],
        out_specs=[_HS_SPEC, _ALAR_SPEC],
        out_shape=[_HS_SHAPE, _ALAR_SHAPE],
    )(aggs, x0s, att2)


def _tc_end(aggs, x0s, wt, b):
    return pl.pallas_call(
        _tc_end_body,
        grid=(N // ROWBLK,),
        in_specs=[
            _HS_SPEC, _HS_SPEC,
            pl.BlockSpec((F, NCLASS), lambda i: (0, 0)),
            pl.BlockSpec((1, NCLASS), lambda i: (0, 0)),
        ],
        out_specs=pl.BlockSpec((ROWBLK, NCLASS), lambda i: (i, 0)),
        out_shape=jax.ShapeDtypeStruct((N, NCLASS), jnp.float32),
    )(aggs, x0s, wt, b)


_MESH = plsc.VectorSubcoreMesh(core_axis_name="c", subcore_axis_name="s")


@functools.partial(
    pl.kernel,
    out_type=jax.ShapeDtypeStruct((NC, N, FH), jnp.float32),
    mesh=_MESH,
    compiler_params=pltpu.CompilerParams(
        needs_layout_passes=False, use_tc_tiling_on_sc=False),
    scratch_types=[
        pltpu.VMEM((2 * N,), jnp.float32),
        pltpu.VMEM((NCHUNK, CH), jnp.int32),
        pltpu.VMEM((NCHUNK, CH), jnp.int32),
        pltpu.VMEM((NCHUNK, CH), jnp.float32),
        pltpu.VMEM((CH, FH), jnp.float32),
        pltpu.VMEM((CH,), jnp.float32),
        pltpu.VMEM_SHARED((N, FH), jnp.float32),
        pltpu.SemaphoreType.DMA,
    ],
)
def _sc_edge(hs_hbm, alar_hbm, src_hbm, dst_hbm, ew_hbm, out_hbm,
             alar_v, src_v, dst_v, ew_v, rows_v, w_v, agg_sh, sem):
    c = lax.axis_index("c")
    s = lax.axis_index("s")

    pltpu.sync_copy(alar_hbm, alar_v)
    pltpu.sync_copy(src_hbm.at[s], src_v)
    pltpu.sync_copy(dst_hbm.at[s], dst_v)
    pltpu.sync_copy(ew_hbm.at[s], ew_v)

    for r in range(CH):
        for j in range(FH // 16):
            rows_v[r, pl.ds(j * 16, 16)] = jnp.zeros((16,), jnp.float32)
    for k in range(RBPT):
        b = s + k * NS

        @pl.when(b < NRB)
        def _():
            off = pl.multiple_of(b * CH, 8)
            pltpu.sync_copy(rows_v, agg_sh.at[pl.ds(off, CH)])

    plsc.subcore_barrier()

    def chunk_body(i, carry):
        pltpu.async_copy(hs_hbm.at[c].at[src_v.at[i]], rows_v, sem).wait()

        for g in range(GPC):
            sidx = src_v[i, pl.ds(g * 16, 16)]
            didx = dst_v[i, pl.ds(g * 16, 16)]
            al = plsc.load_gather(alar_v, [2 * sidx])
            ar = plsc.load_gather(alar_v, [2 * didx + 1])
            sarg = al + ar
            a = jnp.abs(sarg)
            z = jnp.exp(-2.0 * a)
            t = jnp.sign(sarg) * (1.0 - z) / (1.0 + z)
            w = t * ew_v[i, pl.ds(g * 16, 16)]
            for e in range(16):
                r = g * 16 + e
                wb = jnp.broadcast_to(w[e], (16,))
                for j in range(FH // 16):
                    sl = pl.ds(j * 16, 16)
                    rows_v[r, sl] = rows_v[r, sl] * wb

        pltpu.sync_copy(rows_v, agg_sh.at[dst_v.at[i]], add=True)
        return carry

    lax.fori_loop(0, NCHUNK, chunk_body, 0)
    plsc.subcore_barrier()

    for k in range(RBPT):
        b = s + k * NS

        @pl.when(b < NRB)
        def _():
            off = pl.multiple_of(b * CH, 8)
            pltpu.sync_copy(agg_sh.at[pl.ds(off, CH)],
                            out_hbm.at[c, pl.ds(off, CH)])


def kernel(x, edge_index, edge_attr, W_start, b_start, att_l, att_r, W_end, b_end):
    src = edge_index[0].astype(jnp.int32).reshape(NS, NCHUNK, CH)
    dst = edge_index[1].astype(jnp.int32).reshape(NS, NCHUNK, CH)
    ew = edge_attr.reshape(NS, NCHUNK, CH).astype(jnp.float32)

    att20 = jnp.stack([att_l[0], att_r[0]], axis=1)
    att21 = jnp.stack([att_l[1], att_r[1]], axis=1)
    h0s, alar0 = _tc_start(x, W_start.T, b_start.reshape(1, F), att20)
    aggs0 = _sc_edge(h0s, alar0.reshape(2 * N), src, dst, ew)
    h1s, alar1 = _tc_mid(aggs0, h0s, att21)
    aggs1 = _sc_edge(h1s, alar1.reshape(2 * N), src, dst, ew)
    return _tc_end(aggs1, h0s, W_end.T, b_end.reshape(1, NCLASS))


# speedup vs baseline: 22.0177x; 1.5350x over previous
import functools

import jax
import jax.numpy as jnp
from jax import lax
from jax.experimental import pallas as pl
from jax.experimental.pallas import tpu as pltpu
from jax.experimental.pallas import tpu_sc as plsc

N = 10000
F = 128
FH = F // 2
NCLASS = 16
E = 320000
EPS = 0.1

NC = 2
NS = 16
EPT = E // NS
CH = 80
NCHUNK = EPT // CH
GPC = CH // 16
NRB = N // CH
RBPT = -(-NRB // NS)
NBUF = 5
NSUP = NCHUNK // NBUF

ROWBLK = 1000


def _split(h):
    return jnp.stack([h[:, :FH], h[:, FH:]], axis=0)


def _alar(h, att2):
    return jnp.dot(h, att2)


def _tc_start_body(x_ref, wt_ref, b_ref, att2_ref, hs_ref, alar_ref):
    h = jnp.dot(x_ref[...], wt_ref[...], preferred_element_type=jnp.float32)
    h = jnp.maximum(h + b_ref[...], 0.0)
    hs_ref[...] = _split(h)
    alar_ref[...] = _alar(h, att2_ref[...])


def _tc_mid_body(aggs_ref, x0s_ref, att2_ref, hs_ref, alar_ref):
    hs = aggs_ref[...] + EPS * x0s_ref[...]
    hs_ref[...] = hs
    h = jnp.concatenate([hs[0], hs[1]], axis=1)
    alar_ref[...] = _alar(h, att2_ref[...])


def _tc_end_body(aggs_ref, x0s_ref, wt_ref, b_ref, out_ref):
    hs = aggs_ref[...] + EPS * x0s_ref[...]
    h = jnp.concatenate([hs[0], hs[1]], axis=1)
    o = jnp.dot(h, wt_ref[...], preferred_element_type=jnp.float32)
    out_ref[...] = o + b_ref[...]


_HS_SPEC = pl.BlockSpec((NC, ROWBLK, FH), lambda i: (0, i, 0))
_HS_SHAPE = jax.ShapeDtypeStruct((NC, N, FH), jnp.float32)
_ALAR_SPEC = pl.BlockSpec((ROWBLK, 2), lambda i: (i, 0))
_ALAR_SHAPE = jax.ShapeDtypeStruct((N, 2), jnp.float32)
_VEC_SPEC = pl.BlockSpec((1, F), lambda i: (0, 0))
_ATT2_SPEC = pl.BlockSpec((F, 2), lambda i: (0, 0))


def _tc_start(x, wt, b, att2):
    return pl.pallas_call(
        _tc_start_body,
        grid=(N // ROWBLK,),
        in_specs=[
            pl.BlockSpec((ROWBLK, F), lambda i: (i, 0)),
            pl.BlockSpec((F, F), lambda i: (0, 0)),
            _VEC_SPEC, _ATT2_SPEC,
        ],
        out_specs=[_HS_SPEC, _ALAR_SPEC],
        out_shape=[_HS_SHAPE, _ALAR_SHAPE],
    )(x, wt, b, att2)


def _tc_mid(aggs, x0s, att2):
    return pl.pallas_call(
        _tc_mid_body,
        grid=(N // ROWBLK,),
        in_specs=[_HS_SPEC, _HS_SPEC, _ATT2_SPEC],
        out_specs=[_HS_SPEC, _ALAR_SPEC],
        out_shape=[_HS_SHAPE, _ALAR_SHAPE],
    )(aggs, x0s, att2)


def _tc_end(aggs, x0s, wt, b):
    return pl.pallas_call(
        _tc_end_body,
        grid=(N // ROWBLK,),
        in_specs=[
            _HS_SPEC, _HS_SPEC,
            pl.BlockSpec((F, NCLASS), lambda i: (0, 0)),
            pl.BlockSpec((1, NCLASS), lambda i: (0, 0)),
        ],
        out_specs=pl.BlockSpec((ROWBLK, NCLASS), lambda i: (i, 0)),
        out_shape=jax.ShapeDtypeStruct((N, NCLASS), jnp.float32),
    )(aggs, x0s, wt, b)


_MESH = plsc.VectorSubcoreMesh(core_axis_name="c", subcore_axis_name="s")


@functools.partial(
    pl.kernel,
    out_type=jax.ShapeDtypeStruct((NC, N, FH), jnp.float32),
    mesh=_MESH,
    compiler_params=pltpu.CompilerParams(
        needs_layout_passes=False, use_tc_tiling_on_sc=False),
    scratch_types=[
        pltpu.VMEM((2 * N,), jnp.float32),
        pltpu.VMEM((3, NBUF, 3, CH), jnp.int32),
        pltpu.VMEM((CH, FH), jnp.float32),
        pltpu.VMEM((CH, FH), jnp.float32),
        pltpu.VMEM((CH, FH), jnp.float32),
        pltpu.VMEM((CH, FH), jnp.float32),
        pltpu.VMEM((CH, FH), jnp.float32),
        pltpu.VMEM_SHARED((N, FH), jnp.float32),
        pltpu.SemaphoreType.DMA,
        pltpu.SemaphoreType.DMA,
        pltpu.SemaphoreType.DMA,
        pltpu.SemaphoreType.DMA,
        pltpu.SemaphoreType.DMA,
        pltpu.SemaphoreType.DMA,
        pltpu.SemaphoreType.DMA,
        pltpu.SemaphoreType.DMA,
        pltpu.SemaphoreType.DMA,
        pltpu.SemaphoreType.DMA,
        pltpu.SemaphoreType.DMA,
    ],
)
def _sc_edge(hs_hbm, alar_hbm, ep_hbm, out_hbm,
             alar_v, ep_v, rb0, rb1, rb2, rb3, rb4, agg_sh,
             esem, gs0, gs1, gs2, gs3, gs4, ss0, ss1, ss2, ss3, ss4):
    rows_bufs = [rb0, rb1, rb2, rb3, rb4]
    gsems = [gs0, gs1, gs2, gs3, gs4]
    ssems = [ss0, ss1, ss2, ss3, ss4]
    c = lax.axis_index("c")
    s = lax.axis_index("s")

    pltpu.sync_copy(alar_hbm, alar_v)

    zbuf = rows_bufs[0]
    for r in range(CH):
        for j in range(FH // 16):
            zbuf[r, pl.ds(j * 16, 16)] = jnp.zeros((16,), jnp.float32)
    for k in range(RBPT):
        b = s + k * NS

        @pl.when(b < NRB)
        def _():
            off = pl.multiple_of(b * CH, 8)
            pltpu.async_copy(zbuf, agg_sh.at[pl.ds(off, CH)], esem)

    for k in range(RBPT):
        b = s + k * NS

        @pl.when(b < NRB)
        def _():
            off = pl.multiple_of(b * CH, 8)
            pltpu.make_async_copy(zbuf, agg_sh.at[pl.ds(off, CH)], esem).wait()

    plsc.subcore_barrier()

    def scale_chunk(ep_blk, jj, rows_v):
        for g in range(GPC):
            sidx = ep_v[ep_blk, jj, 0, pl.ds(g * 16, 16)]
            didx = ep_v[ep_blk, jj, 1, pl.ds(g * 16, 16)]
            al = plsc.load_gather(alar_v, [2 * sidx])
            ar = plsc.load_gather(alar_v, [2 * didx + 1])
            sarg = al + ar
            a = jnp.abs(sarg)
            z = jnp.exp(-2.0 * a)
            t = jnp.sign(sarg) * (1.0 - z) / (1.0 + z)
            ew = plsc.bitcast(ep_v[ep_blk, jj, 2, pl.ds(g * 16, 16)],
                              jnp.float32)
            w = t * ew
            for e in range(16):
                r = g * 16 + e
                we = w[e]
                for j in range(FH // 16):
                    sl = pl.ds(j * 16, 16)
                    rows_v[r, sl] = rows_v[r, sl] * we

    def stage(sup):
        pltpu.async_copy(ep_hbm.at[s, sup], ep_v.at[lax.rem(sup, 3)], esem)

    def stage_wait():
        pltpu.make_async_copy(ep_hbm.at[s, 0], ep_v.at[0], esem).wait()

    def gather(ep_blk, jj, rows_v, gsem):
        pltpu.async_copy(hs_hbm.at[c].at[ep_v.at[ep_blk, jj, 0]], rows_v, gsem)

    def gather_wait(rows_v, gsem):
        pltpu.make_async_copy(hs_hbm.at[c].at[ep_v.at[0, 0, 0]], rows_v,
                              gsem).wait()

    def scatter(ep_blk, jj, rows_v, ssem):
        pltpu.async_copy(rows_v, agg_sh.at[ep_v.at[ep_blk, jj, 1]], ssem,
                         add=True)

    def scatter_wait(rows_v, ssem):
        pltpu.make_async_copy(rows_v, agg_sh.at[ep_v.at[0, 0, 1]], ssem).wait()

    stage(0)
    stage_wait()
    gather(0, 0, rows_bufs[0], gsems[0])
    gather(0, 1, rows_bufs[1], gsems[1])

    def ring_body(ii, carry):
        p = lax.rem(ii, 3)
        p_next = lax.rem(ii + 1, 3)

        @pl.when(ii + 1 < NSUP)
        def _():
            stage(ii + 1)

        for jj in range(NBUF):
            j = NBUF * ii + jj
            b2 = (jj + 2) % NBUF

            @pl.when(j - 3 >= 0)
            def _():
                scatter_wait(rows_bufs[b2], ssems[b2])

            if jj == 3:
                @pl.when(ii + 1 < NSUP)
                def _():
                    stage_wait()

            @pl.when(j + 2 < NCHUNK)
            def _():
                if jj < 3:
                    gather(p, jj + 2, rows_bufs[b2], gsems[b2])
                else:
                    gather(p_next, jj - 3, rows_bufs[b2], gsems[b2])

            gather_wait(rows_bufs[jj], gsems[jj])
            scale_chunk(p, jj, rows_bufs[jj])
            scatter(p, jj, rows_bufs[jj], ssems[jj])
        return carry

    lax.fori_loop(0, NSUP, ring_body, 0)
    for t in range(3):
        b = (NCHUNK - 3 + t) % NBUF
        scatter_wait(rows_bufs[b], ssems[b])
    plsc.subcore_barrier()

    for k in range(RBPT):
        b = s + k * NS

        @pl.when(b < NRB)
        def _():
            off = pl.multiple_of(b * CH, 8)
            pltpu.async_copy(agg_sh.at[pl.ds(off, CH)],
                             out_hbm.at[c, pl.ds(off, CH)], esem)

    for k in range(RBPT):
        b = s + k * NS

        @pl.when(b < NRB)
        def _():
            off = pl.multiple_of(b * CH, 8)
            pltpu.make_async_copy(agg_sh.at[pl.ds(off, CH)],
                                  out_hbm.at[c, pl.ds(off, CH)], esem).wait()


def kernel(x, edge_index, edge_attr, W_start, b_start, att_l, att_r, W_end, b_end):
    src = edge_index[0].astype(jnp.int32).reshape(NS, NSUP, NBUF, CH)
    dst = edge_index[1].astype(jnp.int32).reshape(NS, NSUP, NBUF, CH)
    ewb = lax.bitcast_convert_type(
        edge_attr.astype(jnp.float32), jnp.int32).reshape(NS, NSUP, NBUF, CH)
    ep = jnp.stack([src, dst, ewb], axis=3)

    att20 = jnp.stack([att_l[0], att_r[0]], axis=1)
    att21 = jnp.stack([att_l[1], att_r[1]], axis=1)
    h0s, alar0 = _tc_start(x, W_start.T, b_start.reshape(1, F), att20)
    aggs0 = _sc_edge(h0s, alar0.reshape(2 * N), ep)
    h1s, alar1 = _tc_mid(aggs0, h0s, att21)
    aggs1 = _sc_edge(h1s, alar1.reshape(2 * N), ep)
    return _tc_end(aggs1, h0s, W_end.T, b_end.reshape(1, NCLASS))


# speedup vs baseline: 22.3620x; 1.0156x over previous
import functools

import jax
import jax.numpy as jnp
from jax import lax
from jax.experimental import pallas as pl
from jax.experimental.pallas import tpu as pltpu
from jax.experimental.pallas import tpu_sc as plsc

N = 10000
F = 128
FH = F // 2
NCLASS = 16
E = 320000
EPS = 0.1

NC = 2
NS = 16
EPT = E // NS
CH = 80
NCHUNK = EPT // CH
GPC = CH // 16
NRB = N // CH
RBPT = -(-NRB // NS)
NBUF = 5
NSUP = NCHUNK // NBUF

ROWBLK = 10000


def _split(h):
    return jnp.stack([h[:, :FH], h[:, FH:]], axis=0)


def _alar(h, att2):
    return jnp.dot(h, att2)


def _tc_start_body(x_ref, wt_ref, b_ref, att2_ref, hs_ref, alar_ref):
    h = jnp.dot(x_ref[...], wt_ref[...], preferred_element_type=jnp.float32)
    h = jnp.maximum(h + b_ref[...], 0.0)
    hs_ref[...] = _split(h)
    alar_ref[...] = _alar(h, att2_ref[...])


def _tc_mid_body(aggs_ref, x0s_ref, att2_ref, hs_ref, alar_ref):
    hs = aggs_ref[...] + EPS * x0s_ref[...]
    hs_ref[...] = hs
    h = jnp.concatenate([hs[0], hs[1]], axis=1)
    alar_ref[...] = _alar(h, att2_ref[...])


def _tc_end_body(aggs_ref, x0s_ref, wt_ref, b_ref, out_ref):
    hs = aggs_ref[...] + EPS * x0s_ref[...]
    h = jnp.concatenate([hs[0], hs[1]], axis=1)
    o = jnp.dot(h, wt_ref[...], preferred_element_type=jnp.float32)
    out_ref[...] = o + b_ref[...]


_HS_SPEC = pl.BlockSpec((NC, ROWBLK, FH), lambda i: (0, i, 0))
_HS_SHAPE = jax.ShapeDtypeStruct((NC, N, FH), jnp.float32)
_ALAR_SPEC = pl.BlockSpec((ROWBLK, 2), lambda i: (i, 0))
_ALAR_SHAPE = jax.ShapeDtypeStruct((N, 2), jnp.float32)
_VEC_SPEC = pl.BlockSpec((1, F), lambda i: (0, 0))
_ATT2_SPEC = pl.BlockSpec((F, 2), lambda i: (0, 0))


def _tc_start(x, wt, b, att2):
    return pl.pallas_call(
        _tc_start_body,
        grid=(N // ROWBLK,),
        in_specs=[
            pl.BlockSpec((ROWBLK, F), lambda i: (i, 0)),
            pl.BlockSpec((F, F), lambda i: (0, 0)),
            _VEC_SPEC, _ATT2_SPEC,
        ],
        out_specs=[_HS_SPEC, _ALAR_SPEC],
        out_shape=[_HS_SHAPE, _ALAR_SHAPE],
    )(x, wt, b, att2)


def _tc_mid(aggs, x0s, att2):
    return pl.pallas_call(
        _tc_mid_body,
        grid=(N // ROWBLK,),
        in_specs=[_HS_SPEC, _HS_SPEC, _ATT2_SPEC],
        out_specs=[_HS_SPEC, _ALAR_SPEC],
        out_shape=[_HS_SHAPE, _ALAR_SHAPE],
    )(aggs, x0s, att2)


def _tc_end(aggs, x0s, wt, b):
    return pl.pallas_call(
        _tc_end_body,
        grid=(N // ROWBLK,),
        in_specs=[
            _HS_SPEC, _HS_SPEC,
            pl.BlockSpec((F, NCLASS), lambda i: (0, 0)),
            pl.BlockSpec((1, NCLASS), lambda i: (0, 0)),
        ],
        out_specs=pl.BlockSpec((ROWBLK, NCLASS), lambda i: (i, 0)),
        out_shape=jax.ShapeDtypeStruct((N, NCLASS), jnp.float32),
    )(aggs, x0s, wt, b)


_MESH = plsc.VectorSubcoreMesh(core_axis_name="c", subcore_axis_name="s")


@functools.partial(
    pl.kernel,
    out_type=jax.ShapeDtypeStruct((NC, N, FH), jnp.float32),
    mesh=_MESH,
    compiler_params=pltpu.CompilerParams(
        needs_layout_passes=False, use_tc_tiling_on_sc=False),
    scratch_types=[
        pltpu.VMEM((2 * N,), jnp.float32),
        pltpu.VMEM((3, NBUF, 3, CH), jnp.int32),
        pltpu.VMEM((CH, FH), jnp.float32),
        pltpu.VMEM((CH, FH), jnp.float32),
        pltpu.VMEM((CH, FH), jnp.float32),
        pltpu.VMEM((CH, FH), jnp.float32),
        pltpu.VMEM((CH, FH), jnp.float32),
        pltpu.VMEM_SHARED((N, FH), jnp.float32),
        pltpu.SemaphoreType.DMA,
        pltpu.SemaphoreType.DMA,
        pltpu.SemaphoreType.DMA,
        pltpu.SemaphoreType.DMA,
        pltpu.SemaphoreType.DMA,
        pltpu.SemaphoreType.DMA,
        pltpu.SemaphoreType.DMA,
        pltpu.SemaphoreType.DMA,
        pltpu.SemaphoreType.DMA,
        pltpu.SemaphoreType.DMA,
        pltpu.SemaphoreType.DMA,
    ],
)
def _sc_edge(hs_hbm, alar_hbm, ep_hbm, out_hbm,
             alar_v, ep_v, rb0, rb1, rb2, rb3, rb4, agg_sh,
             esem, gs0, gs1, gs2, gs3, gs4, ss0, ss1, ss2, ss3, ss4):
    rows_bufs = [rb0, rb1, rb2, rb3, rb4]
    gsems = [gs0, gs1, gs2, gs3, gs4]
    ssems = [ss0, ss1, ss2, ss3, ss4]
    c = lax.axis_index("c")
    s = lax.axis_index("s")

    pltpu.async_copy(alar_hbm, alar_v, gsems[0])
    pltpu.async_copy(ep_hbm.at[s, 0], ep_v.at[0], esem)

    zbuf = rows_bufs[0]
    for r in range(CH):
        for j in range(FH // 16):
            zbuf[r, pl.ds(j * 16, 16)] = jnp.zeros((16,), jnp.float32)
    for k in range(RBPT):
        b = s + k * NS

        @pl.when(b < NRB)
        def _():
            off = pl.multiple_of(b * CH, 8)
            pltpu.async_copy(zbuf, agg_sh.at[pl.ds(off, CH)], ssems[k % NBUF])

    pltpu.make_async_copy(alar_hbm, alar_v, gsems[0]).wait()
    for k in range(RBPT):
        b = s + k * NS

        @pl.when(b < NRB)
        def _():
            off = pl.multiple_of(b * CH, 8)
            pltpu.make_async_copy(zbuf, agg_sh.at[pl.ds(off, CH)],
                                  ssems[k % NBUF]).wait()

    plsc.subcore_barrier()

    def scale_chunk(ep_blk, jj, rows_v):
        for g in range(GPC):
            sidx = ep_v[ep_blk, jj, 0, pl.ds(g * 16, 16)]
            didx = ep_v[ep_blk, jj, 1, pl.ds(g * 16, 16)]
            al = plsc.load_gather(alar_v, [2 * sidx])
            ar = plsc.load_gather(alar_v, [2 * didx + 1])
            sarg = al + ar
            a = jnp.abs(sarg)
            z = jnp.exp(-2.0 * a)
            t = jnp.sign(sarg) * (1.0 - z) / (1.0 + z)
            ew = plsc.bitcast(ep_v[ep_blk, jj, 2, pl.ds(g * 16, 16)],
                              jnp.float32)
            w = t * ew
            for e in range(16):
                r = g * 16 + e
                we = w[e]
                for j in range(FH // 16):
                    sl = pl.ds(j * 16, 16)
                    rows_v[r, sl] = rows_v[r, sl] * we

    def stage(sup):
        pltpu.async_copy(ep_hbm.at[s, sup], ep_v.at[lax.rem(sup, 3)], esem)

    def stage_wait():
        pltpu.make_async_copy(ep_hbm.at[s, 0], ep_v.at[0], esem).wait()

    def gather(ep_blk, jj, rows_v, gsem):
        pltpu.async_copy(hs_hbm.at[c].at[ep_v.at[ep_blk, jj, 0]], rows_v, gsem)

    def gather_wait(rows_v, gsem):
        pltpu.make_async_copy(hs_hbm.at[c].at[ep_v.at[0, 0, 0]], rows_v,
                              gsem).wait()

    def scatter(ep_blk, jj, rows_v, ssem):
        pltpu.async_copy(rows_v, agg_sh.at[ep_v.at[ep_blk, jj, 1]], ssem,
                         add=True)

    def scatter_wait(rows_v, ssem):
        pltpu.make_async_copy(rows_v, agg_sh.at[ep_v.at[0, 0, 1]], ssem).wait()

    stage_wait()
    gather(0, 0, rows_bufs[0], gsems[0])
    gather(0, 1, rows_bufs[1], gsems[1])

    def ring_body(ii, carry):
        p = lax.rem(ii, 3)
        p_next = lax.rem(ii + 1, 3)

        @pl.when(ii + 1 < NSUP)
        def _():
            stage(ii + 1)

        for jj in range(NBUF):
            j = NBUF * ii + jj
            b2 = (jj + 2) % NBUF

            @pl.when(j - 3 >= 0)
            def _():
                scatter_wait(rows_bufs[b2], ssems[b2])

            if jj == 3:
                @pl.when(ii + 1 < NSUP)
                def _():
                    stage_wait()

            @pl.when(j + 2 < NCHUNK)
            def _():
                if jj < 3:
                    gather(p, jj + 2, rows_bufs[b2], gsems[b2])
                else:
                    gather(p_next, jj - 3, rows_bufs[b2], gsems[b2])

            gather_wait(rows_bufs[jj], gsems[jj])
            scale_chunk(p, jj, rows_bufs[jj])
            scatter(p, jj, rows_bufs[jj], ssems[jj])
        return carry

    lax.fori_loop(0, NSUP, ring_body, 0)
    for t in range(3):
        b = (NCHUNK - 3 + t) % NBUF
        scatter_wait(rows_bufs[b], ssems[b])
    plsc.subcore_barrier()

    for k in range(RBPT):
        b = s + k * NS

        @pl.when(b < NRB)
        def _():
            off = pl.multiple_of(b * CH, 8)
            pltpu.async_copy(agg_sh.at[pl.ds(off, CH)],
                             out_hbm.at[c, pl.ds(off, CH)], esem)

    for k in range(RBPT):
        b = s + k * NS

        @pl.when(b < NRB)
        def _():
            off = pl.multiple_of(b * CH, 8)
            pltpu.make_async_copy(agg_sh.at[pl.ds(off, CH)],
                                  out_hbm.at[c, pl.ds(off, CH)], esem).wait()


def kernel(x, edge_index, edge_attr, W_start, b_start, att_l, att_r, W_end, b_end):
    src = edge_index[0].astype(jnp.int32).reshape(NS, NSUP, NBUF, CH)
    dst = edge_index[1].astype(jnp.int32).reshape(NS, NSUP, NBUF, CH)
    ewb = lax.bitcast_convert_type(
        edge_attr.astype(jnp.float32), jnp.int32).reshape(NS, NSUP, NBUF, CH)
    ep = jnp.stack([src, dst, ewb], axis=3)

    att20 = jnp.stack([att_l[0], att_r[0]], axis=1)
    att21 = jnp.stack([att_l[1], att_r[1]], axis=1)
    h0s, alar0 = _tc_start(x, W_start.T, b_start.reshape(1, F), att20)
    aggs0 = _sc_edge(h0s, alar0.reshape(2 * N), ep)
    h1s, alar1 = _tc_mid(aggs0, h0s, att21)
    aggs1 = _sc_edge(h1s, alar1.reshape(2 * N), ep)
    return _tc_end(aggs1, h0s, W_end.T, b_end.reshape(1, NCLASS))
